# initial kernel scaffold (unmeasured)
import jax
import jax.numpy as jnp
from jax import lax
from jax.experimental import pallas as pl
from jax.experimental.pallas import tpu as pltpu

NZ = 4
B, S, D = 2, 512, 2048
DCL = 128
H, DH, DR = 16, 128, 32
BS = B * S

_f32 = jnp.float32


def _dot(a, b):
    return jnp.dot(a, b, preferred_element_type=_f32)


def _gather_kv(x, Wdkv, Wuk, Wuv):

    def body(x_ref, wdkv_ref, wuk_ref, wuv_ref, k_ref, v_ref,
             c_buf, uk_buf, uv_buf, send_sems, recv_sems):
        my_x = lax.axis_index("x")
        my_y = lax.axis_index("y")
        my_z = lax.axis_index("z")
        right = lax.rem(my_z + 1, NZ)

        x2d = x_ref[...].reshape(BS, D)
        c_own = _dot(x2d, wdkv_ref[...])
        c_buf[0] = c_own
        uk_buf[0] = wuk_ref[...]
        uv_buf[0] = wuv_ref[...]

        k_ref[...] = _dot(c_own, wuk_ref[...]).reshape(B, S, D)
        v_ref[...] = _dot(c_own, wuv_ref[...]).reshape(B, S, D)

        for h in range(NZ - 1):
            rdmas = []
            for p, buf in enumerate((c_buf, uk_buf, uv_buf)):
                rdma = pltpu.make_async_remote_copy(
                    src_ref=buf.at[h],
                    dst_ref=buf.at[h + 1],
                    send_sem=send_sems.at[p, h],
                    recv_sem=recv_sems.at[p, h],
                    device_id=(my_x, my_y, right),
                    device_id_type=pl.DeviceIdType.MESH,
                )
                rdma.start()
                rdmas.append(rdma)
            for rdma in rdmas:
                rdma.wait()
            k_ref[...] += _dot(c_buf[h + 1], uk_buf[h + 1]).reshape(B, S, D)
            v_ref[...] += _dot(c_buf[h + 1], uv_buf[h + 1]).reshape(B, S, D)

    return pl.pallas_call(
        body,
        out_shape=[
            jax.ShapeDtypeStruct((B, S, D), _f32),
            jax.ShapeDtypeStruct((B, S, D), _f32),
        ],
        in_specs=[pl.BlockSpec(memory_space=pltpu.VMEM)] * 4,
        out_specs=[pl.BlockSpec(memory_space=pltpu.VMEM)] * 2,
        scratch_shapes=[
            pltpu.VMEM((NZ, BS, DCL), _f32),
            pltpu.VMEM((NZ, DCL, D), _f32),
            pltpu.VMEM((NZ, DCL, D), _f32),
            pltpu.SemaphoreType.DMA((3, NZ - 1)),
            pltpu.SemaphoreType.DMA((3, NZ - 1)),
        ],
        compiler_params=pltpu.CompilerParams(collective_id=0),
    )(x, Wdkv, Wuk, Wuv)


def _q_proj(x, Wq, Wqr, Wkr):
    def body(x_ref, wq_ref, wqr_ref, wkr_ref, q_ref, qr_ref, kr_ref):
        x2d = x_ref[...].reshape(BS, D)
        q_ref[...] = _dot(x2d, wq_ref[...]).reshape(B, S, D)
        qr_ref[...] = _dot(x2d, wqr_ref[...]).reshape(B, S, H * DR)
        kr_ref[...] = _dot(x2d, wkr_ref[...]).reshape(B, S, DR)

    return pl.pallas_call(
        body,
        out_shape=[
            jax.ShapeDtypeStruct((B, S, D), _f32),
            jax.ShapeDtypeStruct((B, S, H * DR), _f32),
            jax.ShapeDtypeStruct((B, S, DR), _f32),
        ],
        in_specs=[pl.BlockSpec(memory_space=pltpu.VMEM)] * 4,
        out_specs=[pl.BlockSpec(memory_space=pltpu.VMEM)] * 3,
    )(x, Wq, Wqr, Wkr)


def _attention(Q, K, V, Qr, Kr):
    scale = (DH + DR) ** -0.5

    def body(q_ref, k_ref, v_ref, qr_ref, kr_ref, o_ref):
        q = q_ref[0]
        k = k_ref[0]
        v = v_ref[0]
        qr = qr_ref[0]
        kr = kr_ref[0]
        s = lax.dot_general(q, k, (((1,), (1,)), ((), ())),
                            preferred_element_type=_f32)
        s += lax.dot_general(qr, kr, (((1,), (1,)), ((), ())),
                             preferred_element_type=_f32)
        s *= scale
        m = jnp.max(s, axis=-1, keepdims=True)
        p = jnp.exp(s - m)
        p /= jnp.sum(p, axis=-1, keepdims=True)
        o_ref[0] = _dot(p, v)

    return pl.pallas_call(
        body,
        grid=(B, H),
        out_shape=jax.ShapeDtypeStruct((B, S, D), _f32),
        in_specs=[
            pl.BlockSpec((1, S, DH), lambda b, h: (b, 0, h)),
            pl.BlockSpec((1, S, DH), lambda b, h: (b, 0, h)),
            pl.BlockSpec((1, S, DH), lambda b, h: (b, 0, h)),
            pl.BlockSpec((1, S, DR), lambda b, h: (b, 0, h)),
            pl.BlockSpec((1, S, DR), lambda b, h: (b, 0, 0)),
        ],
        out_specs=pl.BlockSpec((1, S, DH), lambda b, h: (b, 0, h)),
    )(Q, K, V, Qr, Kr)


def _out_proj(O, Wo):
    def body(o_ref, wo_ref, out_ref):
        out_ref[...] = _dot(o_ref[...].reshape(BS, D), wo_ref[...]).reshape(B, S, D)

    return pl.pallas_call(
        body,
        out_shape=jax.ShapeDtypeStruct((B, S, D), _f32),
        in_specs=[pl.BlockSpec(memory_space=pltpu.VMEM)] * 2,
        out_specs=pl.BlockSpec(memory_space=pltpu.VMEM),
    )(O, Wo)


def kernel(x, Wdkv, Wuk, Wuv, Wq, Wqr, Wkr, Wo):
    K, V = _gather_kv(x, Wdkv, Wuk, Wuv)
    Q, Qr, Kr = _q_proj(x, Wq, Wqr, Wkr)
    O = _attention(Q, K, V, Qr, Kr)
    return _out_proj(O, Wo)


# baseline (device time: 218197 ns/iter reference)
import jax
import jax.numpy as jnp
from jax import lax
from jax.experimental import pallas as pl
from jax.experimental.pallas import tpu as pltpu

NZ = 4
B, S, D = 2, 512, 2048
DCL = 128
H, DH, DR = 16, 128, 32
BS = B * S

_f32 = jnp.float32


def _dot(a, b):
    return jnp.dot(a, b, preferred_element_type=_f32)


def _gather_kv(x, Wdkv, Wuk, Wuv):

    def body(x_ref, wdkv_ref, wuk_ref, wuv_ref, k_ref, v_ref,
             c_buf, uk_buf, uv_buf, send_sems, recv_sems):
        my_x = lax.axis_index("x")
        my_y = lax.axis_index("y")
        my_z = lax.axis_index("z")
        right = lax.rem(my_z + 1, NZ)

        x2d = x_ref[...].reshape(BS, D)
        c_own = _dot(x2d, wdkv_ref[...])
        c_buf[0] = c_own
        uk_buf[0] = wuk_ref[...]
        uv_buf[0] = wuv_ref[...]

        k_ref[...] = _dot(c_own, wuk_ref[...]).reshape(B, S, D)
        v_ref[...] = _dot(c_own, wuv_ref[...]).reshape(B, S, D)

        for h in range(NZ - 1):
            rdmas = []
            for p, buf in enumerate((c_buf, uk_buf, uv_buf)):
                rdma = pltpu.make_async_remote_copy(
                    src_ref=buf.at[h],
                    dst_ref=buf.at[h + 1],
                    send_sem=send_sems.at[p, h],
                    recv_sem=recv_sems.at[p, h],
                    device_id=(my_x, my_y, right),
                    device_id_type=pl.DeviceIdType.MESH,
                )
                rdma.start()
                rdmas.append(rdma)
            for rdma in rdmas:
                rdma.wait()
            k_ref[...] += _dot(c_buf[h + 1], uk_buf[h + 1]).reshape(B, S, D)
            v_ref[...] += _dot(c_buf[h + 1], uv_buf[h + 1]).reshape(B, S, D)

    return pl.pallas_call(
        body,
        out_shape=[
            jax.ShapeDtypeStruct((B, S, D), _f32),
            jax.ShapeDtypeStruct((B, S, D), _f32),
        ],
        in_specs=[pl.BlockSpec(memory_space=pltpu.VMEM)] * 4,
        out_specs=[pl.BlockSpec(memory_space=pltpu.VMEM)] * 2,
        scratch_shapes=[
            pltpu.VMEM((NZ, BS, DCL), _f32),
            pltpu.VMEM((NZ, DCL, D), _f32),
            pltpu.VMEM((NZ, DCL, D), _f32),
            pltpu.SemaphoreType.DMA((3, NZ - 1)),
            pltpu.SemaphoreType.DMA((3, NZ - 1)),
        ],
    )(x, Wdkv, Wuk, Wuv)


def _q_proj(x, Wq, Wqr, Wkr):
    def body(x_ref, wq_ref, wqr_ref, wkr_ref, q_ref, qr_ref, kr_ref):
        x2d = x_ref[...].reshape(BS, D)
        q_ref[...] = _dot(x2d, wq_ref[...]).reshape(B, S, D)
        qr4 = _dot(x2d, wqr_ref[...]).reshape(B, S, H, DR)
        qr_ref[...] = jnp.transpose(qr4, (0, 2, 1, 3))
        kr_ref[...] = _dot(x2d, wkr_ref[...]).reshape(B, S, DR)

    return pl.pallas_call(
        body,
        out_shape=[
            jax.ShapeDtypeStruct((B, S, D), _f32),
            jax.ShapeDtypeStruct((B, H, S, DR), _f32),
            jax.ShapeDtypeStruct((B, S, DR), _f32),
        ],
        in_specs=[pl.BlockSpec(memory_space=pltpu.VMEM)] * 4,
        out_specs=[pl.BlockSpec(memory_space=pltpu.VMEM)] * 3,
    )(x, Wq, Wqr, Wkr)


def _attention(Q, K, V, Qr, Kr):
    scale = (DH + DR) ** -0.5

    def body(q_ref, k_ref, v_ref, qr_ref, kr_ref, o_ref):
        q = q_ref[0]
        k = k_ref[0]
        v = v_ref[0]
        qr = qr_ref[0, 0]
        kr = kr_ref[0]
        s = lax.dot_general(q, k, (((1,), (1,)), ((), ())),
                            preferred_element_type=_f32)
        s += lax.dot_general(qr, kr, (((1,), (1,)), ((), ())),
                             preferred_element_type=_f32)
        s *= scale
        m = jnp.max(s, axis=-1, keepdims=True)
        p = jnp.exp(s - m)
        p /= jnp.sum(p, axis=-1, keepdims=True)
        o_ref[0] = _dot(p, v)

    return pl.pallas_call(
        body,
        grid=(B, H),
        out_shape=jax.ShapeDtypeStruct((B, S, D), _f32),
        in_specs=[
            pl.BlockSpec((1, S, DH), lambda b, h: (b, 0, h)),
            pl.BlockSpec((1, S, DH), lambda b, h: (b, 0, h)),
            pl.BlockSpec((1, S, DH), lambda b, h: (b, 0, h)),
            pl.BlockSpec((1, 1, S, DR), lambda b, h: (b, h, 0, 0)),
            pl.BlockSpec((1, S, DR), lambda b, h: (b, 0, 0)),
        ],
        out_specs=pl.BlockSpec((1, S, DH), lambda b, h: (b, 0, h)),
    )(Q, K, V, Qr, Kr)


def _out_proj(O, Wo):
    def body(o_ref, wo_ref, out_ref):
        out_ref[...] = _dot(o_ref[...].reshape(BS, D), wo_ref[...]).reshape(B, S, D)

    return pl.pallas_call(
        body,
        out_shape=jax.ShapeDtypeStruct((B, S, D), _f32),
        in_specs=[pl.BlockSpec(memory_space=pltpu.VMEM)] * 2,
        out_specs=pl.BlockSpec(memory_space=pltpu.VMEM),
    )(O, Wo)


def kernel(x, Wdkv, Wuk, Wuv, Wq, Wqr, Wkr, Wo):
    K, V = _gather_kv(x, Wdkv, Wuk, Wuv)
    Q, Qr, Kr = _q_proj(x, Wq, Wqr, Wkr)
    O = _attention(Q, K, V, Qr, Kr)
    return _out_proj(O, Wo)
